# baseline (speedup 1.0000x reference)
import jax
import jax.numpy as jnp
from jax import lax
from jax.experimental import pallas as pl
from jax.experimental.pallas import tpu as pltpu
from jax.experimental.pallas import tpu_sc as plsc

N = 10000
K = 2500
D = 128

CHUNK = 128
NSUB = 16
STEPS = 5
SPAN = STEPS * CHUNK
LSPAN = N - 15 * SPAN
LFULL = LSPAN // CHUNK
LTAIL = LSPAN - LFULL * CHUNK
ACC_ROWS = 2560
TRASH = ACC_ROWS - 1
ZROWS = ACC_ROWS // NSUB
OTAIL = K - 15 * ZROWS


def _body(x_hbm, idx_hbm, out_hbm, idx2, xbuf, zbuf, acc,
          sem_i, sem_x, sem_s):
    s = lax.axis_index("s")
    base = s * SPAN

    @pl.when(s < 15)
    def _():
        for t in range(STEPS):
            pltpu.async_copy(idx_hbm.at[pl.ds(base + t * CHUNK, CHUNK)],
                             idx2.at[t], sem_i.at[t])
        pltpu.async_copy(x_hbm.at[pl.ds(base, SPAN)],
                         xbuf.at[pl.ds(0, SPAN)], sem_x)

    @pl.when(s == 15)
    def _():
        for t in range(LFULL):
            pltpu.async_copy(idx_hbm.at[pl.ds(base + t * CHUNK, CHUNK)],
                             idx2.at[t], sem_i.at[t])
        pltpu.async_copy(idx_hbm.at[pl.ds(base + LFULL * CHUNK, LTAIL)],
                         idx2.at[LFULL, pl.ds(0, LTAIL)], sem_i.at[LFULL])
        pltpu.async_copy(x_hbm.at[pl.ds(base, LSPAN)],
                         xbuf.at[pl.ds(0, LSPAN)], sem_x)
        for q in range(LTAIL // 16, CHUNK // 16):
            idx2[LFULL, pl.ds(16 * q, 16)] = jnp.full((16,), TRASH, jnp.int32)

    def _zrow(i, carry):
        for g in range(D // 16):
            zbuf[i, pl.ds(16 * g, 16)] = jnp.zeros((16,), jnp.float32)
        return carry

    lax.fori_loop(0, ZROWS, _zrow, 0)
    pltpu.sync_copy(zbuf, acc.at[pl.ds(s * ZROWS, ZROWS)])
    plsc.subcore_barrier()

    @pl.when(s < 15)
    def _():
        pltpu.make_async_copy(x_hbm.at[pl.ds(base, SPAN)],
                              xbuf.at[pl.ds(0, SPAN)], sem_x).wait()
        for t in range(STEPS):
            pltpu.make_async_copy(idx_hbm.at[pl.ds(0, CHUNK)],
                                  idx2.at[t], sem_i.at[t]).wait()
            pltpu.async_copy(xbuf.at[pl.ds(t * CHUNK, CHUNK)],
                             acc.at[idx2.at[t]], sem_s.at[t], add=True)
        for t in range(STEPS):
            pltpu.make_async_copy(xbuf.at[pl.ds(t * CHUNK, CHUNK)],
                                  acc.at[idx2.at[t]], sem_s.at[t]).wait()

    @pl.when(s == 15)
    def _():
        pltpu.make_async_copy(x_hbm.at[pl.ds(base, LSPAN)],
                              xbuf.at[pl.ds(0, LSPAN)], sem_x).wait()
        pltpu.make_async_copy(idx_hbm.at[pl.ds(0, LTAIL)],
                              idx2.at[LFULL, pl.ds(0, LTAIL)],
                              sem_i.at[LFULL]).wait()
        for t in range(LFULL):
            pltpu.make_async_copy(idx_hbm.at[pl.ds(0, CHUNK)],
                                  idx2.at[t], sem_i.at[t]).wait()
        for t in range(LFULL + 1):
            pltpu.async_copy(xbuf.at[pl.ds(t * CHUNK, CHUNK)],
                             acc.at[idx2.at[t]], sem_s.at[t], add=True)
        for t in range(LFULL + 1):
            pltpu.make_async_copy(xbuf.at[pl.ds(t * CHUNK, CHUNK)],
                                  acc.at[idx2.at[t]], sem_s.at[t]).wait()

    plsc.subcore_barrier()

    @pl.when(s < 15)
    def _():
        pltpu.sync_copy(acc.at[pl.ds(s * ZROWS, ZROWS)],
                        out_hbm.at[pl.ds(s * ZROWS, ZROWS)])

    @pl.when(s == 15)
    def _():
        pltpu.sync_copy(acc.at[pl.ds(15 * ZROWS, OTAIL)],
                        out_hbm.at[pl.ds(15 * ZROWS, OTAIL)])


@jax.jit
def _segment_sum_sc(x, index):
    mesh = plsc.VectorSubcoreMesh(core_axis_name="c", subcore_axis_name="s",
                                  num_cores=1)
    f = pl.kernel(
        _body,
        out_type=jax.ShapeDtypeStruct((K, D), jnp.float32),
        mesh=mesh,
        scratch_types=[
            pltpu.VMEM((STEPS, CHUNK), jnp.int32),
            pltpu.VMEM((SPAN, D), jnp.float32),
            pltpu.VMEM((ZROWS, D), jnp.float32),
            pltpu.VMEM_SHARED((ACC_ROWS, D), jnp.float32),
            pltpu.SemaphoreType.DMA((STEPS,)),
            pltpu.SemaphoreType.DMA,
            pltpu.SemaphoreType.DMA((STEPS,)),
        ],
    )
    return f(x, index)


def kernel(x, adj, index, W, b):
    del adj, W, b
    return _segment_sum_sc(x, index)


# speedup vs baseline: 1.0316x; 1.0316x over previous
import jax
import jax.numpy as jnp
from jax import lax
from jax.experimental import pallas as pl
from jax.experimental.pallas import tpu as pltpu
from jax.experimental.pallas import tpu_sc as plsc

N = 10000
K = 2500
D = 128

CHUNK = 128
NFULL = N // CHUNK
TAIL = N - NFULL * CHUNK
NSUB = 16
STEPS = 5
ACC_ROWS = 2560
TRASH = ACC_ROWS - 1
ZROWS = ACC_ROWS // NSUB
OTAIL = K - 15 * ZROWS


def _body(x_hbm, idx_hbm, out_hbm, idx2, xbuf, zbuf, acc, sem_i, sem_x,
          sem_s):
    s = lax.axis_index("s")

    for t in range(STEPS):
        j = s + NSUB * t

        @pl.when(j < NFULL)
        def _():
            pltpu.async_copy(idx_hbm.at[pl.ds(j * CHUNK, CHUNK)],
                             idx2.at[t], sem_i.at[t])
            pltpu.async_copy(x_hbm.at[pl.ds(j * CHUNK, CHUNK)],
                             xbuf.at[t], sem_x.at[t])

        @pl.when(j == NFULL + 1)
        def _():
            pltpu.async_copy(idx_hbm.at[pl.ds(NFULL * CHUNK, TAIL)],
                             idx2.at[STEPS - 1, pl.ds(0, TAIL)],
                             sem_i.at[STEPS - 1])
            pltpu.async_copy(x_hbm.at[pl.ds(NFULL * CHUNK, TAIL)],
                             xbuf.at[STEPS - 1, pl.ds(0, TAIL)],
                             sem_x.at[STEPS - 1])

    def _zrow(i, carry):
        for g in range(D // 16):
            zbuf[i, pl.ds(16 * g, 16)] = jnp.zeros((16,), jnp.float32)
        return carry

    lax.fori_loop(0, ZROWS, _zrow, 0)
    pltpu.sync_copy(zbuf, acc.at[pl.ds(s * ZROWS, ZROWS)])
    plsc.subcore_barrier()

    for t in range(STEPS):
        j = s + NSUB * t

        @pl.when(j < NFULL)
        def _():
            pltpu.make_async_copy(idx_hbm.at[pl.ds(j * CHUNK, CHUNK)],
                                  idx2.at[t], sem_i.at[t]).wait()
            pltpu.make_async_copy(x_hbm.at[pl.ds(j * CHUNK, CHUNK)],
                                  xbuf.at[t], sem_x.at[t]).wait()
            pltpu.async_copy(xbuf.at[t], acc.at[idx2.at[t]],
                             sem_s.at[t], add=True)

        @pl.when(j == NFULL + 1)
        def _():
            pltpu.make_async_copy(idx_hbm.at[pl.ds(NFULL * CHUNK, TAIL)],
                                  idx2.at[STEPS - 1, pl.ds(0, TAIL)],
                                  sem_i.at[STEPS - 1]).wait()
            pltpu.make_async_copy(x_hbm.at[pl.ds(NFULL * CHUNK, TAIL)],
                                  xbuf.at[STEPS - 1, pl.ds(0, TAIL)],
                                  sem_x.at[STEPS - 1]).wait()
            for q in range(TAIL // 16, CHUNK // 16):
                idx2[STEPS - 1, pl.ds(16 * q, 16)] = jnp.full(
                    (16,), TRASH, jnp.int32)
            pltpu.async_copy(xbuf.at[STEPS - 1], acc.at[idx2.at[STEPS - 1]],
                             sem_s.at[STEPS - 1], add=True)

    for t in range(STEPS):
        j = s + NSUB * t

        @pl.when((j < NFULL) | (j == NFULL + 1))
        def _():
            pltpu.make_async_copy(xbuf.at[t], acc.at[idx2.at[t]],
                                  sem_s.at[t]).wait()

    plsc.subcore_barrier()

    @pl.when(s < 15)
    def _():
        pltpu.sync_copy(acc.at[pl.ds(s * ZROWS, ZROWS)],
                        out_hbm.at[pl.ds(s * ZROWS, ZROWS)])

    @pl.when(s == 15)
    def _():
        pltpu.sync_copy(acc.at[pl.ds(15 * ZROWS, OTAIL)],
                        out_hbm.at[pl.ds(15 * ZROWS, OTAIL)])


@jax.jit
def _segment_sum_sc(x, index):
    mesh = plsc.VectorSubcoreMesh(core_axis_name="c", subcore_axis_name="s",
                                  num_cores=1)
    f = pl.kernel(
        _body,
        out_type=jax.ShapeDtypeStruct((K, D), jnp.float32),
        mesh=mesh,
        scratch_types=[
            pltpu.VMEM((STEPS, CHUNK), jnp.int32),
            pltpu.VMEM((STEPS, CHUNK, D), jnp.float32),
            pltpu.VMEM((ZROWS, D), jnp.float32),
            pltpu.VMEM_SHARED((ACC_ROWS, D), jnp.float32),
            pltpu.SemaphoreType.DMA((STEPS,)),
            pltpu.SemaphoreType.DMA((STEPS,)),
            pltpu.SemaphoreType.DMA((STEPS,)),
        ],
    )
    return f(x, index)


def kernel(x, adj, index, W, b):
    del adj, W, b
    return _segment_sum_sc(x, index)


# speedup vs baseline: 1.0369x; 1.0052x over previous
import jax
import jax.numpy as jnp
from jax import lax
from jax.experimental import pallas as pl
from jax.experimental.pallas import tpu as pltpu
from jax.experimental.pallas import tpu_sc as plsc

N = 10000
K = 2500
D = 128

CHUNK = 128
NFULL = N // CHUNK
TAIL = N - NFULL * CHUNK
NSUB = 16
STEPS = 5
ACC_ROWS = 2560
TRASH = ACC_ROWS - 1
ZROWS = ACC_ROWS // NSUB
OTAIL = K - 15 * ZROWS
ZBUF_ROWS = 16


def _body(x_hbm, idx_hbm, out_hbm, idx2, xbuf, zbuf, acc, sem_i, sem_x,
          sem_s, sem_z):
    s = lax.axis_index("s")

    for t in range(STEPS):
        j = s + NSUB * t

        @pl.when(j < NFULL)
        def _():
            pltpu.async_copy(idx_hbm.at[pl.ds(j * CHUNK, CHUNK)],
                             idx2.at[t], sem_i.at[t])
            pltpu.async_copy(x_hbm.at[pl.ds(j * CHUNK, CHUNK)],
                             xbuf.at[t], sem_x.at[t])

        @pl.when(j == NFULL + 1)
        def _():
            pltpu.async_copy(idx_hbm.at[pl.ds(NFULL * CHUNK, TAIL)],
                             idx2.at[STEPS - 1, pl.ds(0, TAIL)],
                             sem_i.at[STEPS - 1])
            pltpu.async_copy(x_hbm.at[pl.ds(NFULL * CHUNK, TAIL)],
                             xbuf.at[STEPS - 1, pl.ds(0, TAIL)],
                             sem_x.at[STEPS - 1])

    def _zrow(i, carry):
        for g in range(D // 16):
            zbuf[i, pl.ds(16 * g, 16)] = jnp.zeros((16,), jnp.float32)
        return carry

    lax.fori_loop(0, ZBUF_ROWS, _zrow, 0)
    for r in range(ZROWS // ZBUF_ROWS):
        pltpu.async_copy(
            zbuf, acc.at[pl.ds(s * ZROWS + r * ZBUF_ROWS, ZBUF_ROWS)], sem_z)
    for r in range(ZROWS // ZBUF_ROWS):
        pltpu.make_async_copy(
            zbuf, acc.at[pl.ds(s * ZROWS + r * ZBUF_ROWS, ZBUF_ROWS)],
            sem_z).wait()
    plsc.subcore_barrier()

    for t in range(STEPS):
        j = s + NSUB * t

        @pl.when(j < NFULL)
        def _():
            pltpu.make_async_copy(idx_hbm.at[pl.ds(j * CHUNK, CHUNK)],
                                  idx2.at[t], sem_i.at[t]).wait()
            pltpu.make_async_copy(x_hbm.at[pl.ds(j * CHUNK, CHUNK)],
                                  xbuf.at[t], sem_x.at[t]).wait()
            pltpu.async_copy(xbuf.at[t], acc.at[idx2.at[t]],
                             sem_s.at[t], add=True)

        @pl.when(j == NFULL + 1)
        def _():
            pltpu.make_async_copy(idx_hbm.at[pl.ds(NFULL * CHUNK, TAIL)],
                                  idx2.at[STEPS - 1, pl.ds(0, TAIL)],
                                  sem_i.at[STEPS - 1]).wait()
            pltpu.make_async_copy(x_hbm.at[pl.ds(NFULL * CHUNK, TAIL)],
                                  xbuf.at[STEPS - 1, pl.ds(0, TAIL)],
                                  sem_x.at[STEPS - 1]).wait()
            for q in range(TAIL // 16, CHUNK // 16):
                idx2[STEPS - 1, pl.ds(16 * q, 16)] = jnp.full(
                    (16,), TRASH, jnp.int32)
            pltpu.async_copy(xbuf.at[STEPS - 1], acc.at[idx2.at[STEPS - 1]],
                             sem_s.at[STEPS - 1], add=True)

    for t in range(STEPS):
        j = s + NSUB * t

        @pl.when((j < NFULL) | (j == NFULL + 1))
        def _():
            pltpu.make_async_copy(xbuf.at[t], acc.at[idx2.at[t]],
                                  sem_s.at[t]).wait()

    plsc.subcore_barrier()

    @pl.when(s < 15)
    def _():
        pltpu.sync_copy(acc.at[pl.ds(s * ZROWS, ZROWS)],
                        out_hbm.at[pl.ds(s * ZROWS, ZROWS)])

    @pl.when(s == 15)
    def _():
        pltpu.sync_copy(acc.at[pl.ds(15 * ZROWS, OTAIL)],
                        out_hbm.at[pl.ds(15 * ZROWS, OTAIL)])


@jax.jit
def _segment_sum_sc(x, index):
    mesh = plsc.VectorSubcoreMesh(core_axis_name="c", subcore_axis_name="s",
                                  num_cores=1)
    f = pl.kernel(
        _body,
        out_type=jax.ShapeDtypeStruct((K, D), jnp.float32),
        mesh=mesh,
        scratch_types=[
            pltpu.VMEM((STEPS, CHUNK), jnp.int32),
            pltpu.VMEM((STEPS, CHUNK, D), jnp.float32),
            pltpu.VMEM((ZBUF_ROWS, D), jnp.float32),
            pltpu.VMEM_SHARED((ACC_ROWS, D), jnp.float32),
            pltpu.SemaphoreType.DMA((STEPS,)),
            pltpu.SemaphoreType.DMA((STEPS,)),
            pltpu.SemaphoreType.DMA((STEPS,)),
            pltpu.SemaphoreType.DMA,
        ],
    )
    return f(x, index)


def kernel(x, adj, index, W, b):
    del adj, W, b
    return _segment_sum_sc(x, index)
